# Initial kernel scaffold; baseline (speedup 1.0000x reference)
import jax
import jax.numpy as jnp
from jax.experimental import pallas as pl


def kernel(inputs, edge_index, W1, b1, W2, b2, W3, b3, W4, b4, W5, b5, W6, b6):
    raise NotImplementedError("write your pallas kernel here")


# speedup vs baseline: 3.3399x; 3.3399x over previous
import functools

import jax
import jax.numpy as jnp
from jax import lax
from jax.experimental import pallas as pl
from jax.experimental.pallas import tpu as pltpu
from jax.experimental.pallas import tpu_sc as plsc

N = 10000
E = 320000
D = 128

NC = 2
NS = 16
NW = NC * NS
EPW = E // NW
EPT = 10240
BATCH = 80
NBATCH = EPT // BATCH
PAIRS = NBATCH // 2
ADDS = BATCH // 16
NP = 10112
RPT = NP // NS

_mesh = plsc.VectorSubcoreMesh(core_axis_name="c", subcore_axis_name="s")
_sc_params = pltpu.CompilerParams(needs_layout_passes=False)


@functools.partial(
    pl.kernel,
    mesh=_mesh,
    out_type=jax.ShapeDtypeStruct((NW, 2, N), jnp.float32),
    name="gcn_degrees",
    compiler_params=_sc_params,
    scratch_types=[
        pltpu.VMEM((EPT,), jnp.int32),
        pltpu.VMEM((EPT,), jnp.int32),
        pltpu.VMEM((2, N), jnp.float32),
    ],
)
def _deg_kernel(src_hbm, dst_hbm, out_hbm, sidx, didx, deg2):
    c = lax.axis_index("c")
    s = lax.axis_index("s")
    wid = c * NS + s
    base = wid * EPT
    pltpu.sync_copy(src_hbm.at[pl.ds(base, EPT)], sidx)
    pltpu.sync_copy(dst_hbm.at[pl.ds(base, EPT)], didx)

    zeros16 = jnp.zeros((16,), jnp.float32)

    def zero_body(i, _):
        deg2[0, pl.ds(i * 16, 16)] = zeros16
        deg2[1, pl.ds(i * 16, 16)] = zeros16
        return 0

    lax.fori_loop(0, N // 16, zero_body, 0)

    ones16 = jnp.ones((16,), jnp.float32)
    zi16 = jnp.zeros((16,), jnp.int32)
    oi16 = jnp.ones((16,), jnp.int32)

    def count_body(j, _):
        s16 = sidx[pl.ds(j * 16, 16)]
        d16 = didx[pl.ds(j * 16, 16)]
        plsc.addupdate_scatter(deg2, [zi16, s16], ones16)
        plsc.addupdate_scatter(deg2, [oi16, d16], ones16)
        return 0

    lax.fori_loop(0, EPW // 16, count_body, 0)

    pltpu.sync_copy(deg2, out_hbm.at[wid])


@functools.partial(
    pl.kernel,
    mesh=_mesh,
    out_type=jax.ShapeDtypeStruct((NC, NP, D), jnp.float32),
    name="gcn_aggregate",
    compiler_params=_sc_params,
    scratch_types=[
        pltpu.VMEM((EPT,), jnp.int32),
        pltpu.VMEM((EPT,), jnp.int32),
        pltpu.VMEM((BATCH, D), jnp.float32),
        pltpu.VMEM((BATCH, D), jnp.float32),
        pltpu.VMEM_SHARED((NP, D), jnp.float32),
        pltpu.SemaphoreType.DMA,
        pltpu.SemaphoreType.DMA,
        pltpu.SemaphoreType.DMA,
        pltpu.SemaphoreType.DMA,
    ],
)
def _agg_kernel(h_hbm, src_hbm, dst_hbm, zeros_hbm, out_hbm,
                sidx, didx, rows0, rows1, acc, sem0, sem1, semA0, semA1):
    c = lax.axis_index("c")
    s = lax.axis_index("s")
    wid = c * NS + s
    base = wid * EPT

    pltpu.sync_copy(src_hbm.at[pl.ds(base, EPT)], sidx)
    pltpu.sync_copy(dst_hbm.at[pl.ds(base, EPT)], didx)
    pltpu.sync_copy(zeros_hbm, acc.at[pl.ds(s * RPT, RPT)])
    plsc.subcore_barrier()

    def gather_desc(j, buf, sem):
        return pltpu.make_async_copy(
            h_hbm.at[sidx.at[pl.ds(j * BATCH, BATCH)]], buf, sem)

    def fire_adds(j, buf, sem):
        for o in range(ADDS):
            d16 = didx[pl.ds(j * BATCH + o * 16, 16)]
            pltpu.async_copy(buf.at[pl.ds(o * 16, 16)], acc.at[d16],
                             sem, add=True)

    def drain_adds(buf, sem):
        pltpu.make_async_copy(zeros_hbm.at[pl.ds(0, BATCH)], buf, sem).wait()

    gather_desc(0, rows0, sem0).start()

    def body(g, _):
        j0 = 2 * g
        j1 = j0 + 1
        gather_desc(j0, rows0, sem0).wait()
        fire_adds(j0, rows0, semA0)

        @pl.when(g > 0)
        def _():
            drain_adds(rows1, semA1)

        gather_desc(j1, rows1, sem1).start()
        gather_desc(j1, rows1, sem1).wait()
        fire_adds(j1, rows1, semA1)
        drain_adds(rows0, semA0)

        @pl.when(g < PAIRS - 1)
        def _():
            gather_desc(j0 + 2, rows0, sem0).start()

        return 0

    lax.fori_loop(0, PAIRS, body, 0)
    drain_adds(rows1, semA1)

    plsc.subcore_barrier()
    pltpu.sync_copy(acc.at[pl.ds(s * RPT, RPT)],
                    out_hbm.at[c, pl.ds(s * RPT, RPT)])


_BN = 128
_GRID = NP // _BN


def _prep_body(degs_ref, x_ref, h0s_ref, nin_ref, nout_ref):
    degs = degs_ref[...]
    dego = jnp.sum(degs[:, 0, :], axis=0)
    degi = jnp.sum(degs[:, 1, :], axis=0)
    nout = lax.rsqrt(jnp.maximum(dego, 1.0))
    nin = lax.rsqrt(jnp.maximum(degi, 1.0))
    h0s_ref[...] = x_ref[...] * nout[:, None]
    nin_ref[...] = nin[:, None]
    nout_ref[...] = nout[:, None]


_prep_call = pl.pallas_call(
    _prep_body,
    grid=(_GRID,),
    in_specs=[
        pl.BlockSpec((NW, 2, _BN), lambda i: (0, 0, i)),
        pl.BlockSpec((_BN, D), lambda i: (i, 0)),
    ],
    out_specs=[
        pl.BlockSpec((_BN, D), lambda i: (i, 0)),
        pl.BlockSpec((_BN, 1), lambda i: (i, 0)),
        pl.BlockSpec((_BN, 1), lambda i: (i, 0)),
    ],
    out_shape=[
        jax.ShapeDtypeStruct((NP, D), jnp.float32),
        jax.ShapeDtypeStruct((N, 1), jnp.float32),
        jax.ShapeDtypeStruct((N, 1), jnp.float32),
    ],
)


def _layer_body_mid(agg_ref, nin_ref, nout_ref, w_ref, b_ref, h_ref, hs_ref):
    a = (agg_ref[0] + agg_ref[1]) * nin_ref[...]
    z = jnp.dot(a, w_ref[...], preferred_element_type=jnp.float32) + b_ref[...]
    h = jnp.maximum(z, 0.0)
    h_ref[...] = h
    hs_ref[...] = h * nout_ref[...]


def _layer_body_last(agg_ref, nin_ref, nout_ref, w_ref, b_ref, h_ref):
    a = (agg_ref[0] + agg_ref[1]) * nin_ref[...]
    z = jnp.dot(a, w_ref[...], preferred_element_type=jnp.float32) + b_ref[...]
    h_ref[...] = z


_layer_in_specs = [
    pl.BlockSpec((NC, _BN, D), lambda i: (0, i, 0)),
    pl.BlockSpec((_BN, 1), lambda i: (i, 0)),
    pl.BlockSpec((_BN, 1), lambda i: (i, 0)),
    pl.BlockSpec((D, D), lambda i: (0, 0)),
    pl.BlockSpec((1, D), lambda i: (0, 0)),
]

_layer_call_mid = pl.pallas_call(
    _layer_body_mid,
    grid=(_GRID,),
    in_specs=_layer_in_specs,
    out_specs=[
        pl.BlockSpec((_BN, D), lambda i: (i, 0)),
        pl.BlockSpec((_BN, D), lambda i: (i, 0)),
    ],
    out_shape=[
        jax.ShapeDtypeStruct((N, D), jnp.float32),
        jax.ShapeDtypeStruct((NP, D), jnp.float32),
    ],
)

_layer_call_last = pl.pallas_call(
    _layer_body_last,
    grid=(_GRID,),
    in_specs=_layer_in_specs,
    out_specs=pl.BlockSpec((_BN, D), lambda i: (i, 0)),
    out_shape=jax.ShapeDtypeStruct((N, D), jnp.float32),
)


@jax.jit
def _run(inputs, edge_index, ws, bs):
    ei = edge_index.astype(jnp.int32).reshape(2, NW, EPW)
    ei = jnp.pad(ei, ((0, 0), (0, 0), (0, EPT - EPW)), constant_values=N)
    src = ei[0].reshape(NW * EPT)
    dst = ei[1].reshape(NW * EPT)

    degs = _deg_kernel(src, dst)
    h_scaled, nin, nout = _prep_call(degs, inputs)
    zeros = jnp.zeros((RPT, D), jnp.float32)

    hs = []
    for i in range(6):
        agg = _agg_kernel(h_scaled, src, dst, zeros)
        b2 = bs[i].reshape(1, D)
        if i < 5:
            h, h_scaled = _layer_call_mid(agg, nin, nout, ws[i], b2)
        else:
            h = _layer_call_last(agg, nin, nout, ws[i], b2)
        hs.append(h)
    return (hs[5], hs[4], hs[3], hs[2], hs[1], hs[0])


def kernel(inputs, edge_index, W1, b1, W2, b2, W3, b3, W4, b4, W5, b5, W6, b6):
    return _run(inputs, edge_index,
                (W1, W2, W3, W4, W5, W6), (b1, b2, b3, b4, b5, b6))
